# baseline (speedup 1.0000x reference)
import functools

import jax
import jax.numpy as jnp
from jax import lax
from jax.experimental import pallas as pl
from jax.experimental.pallas import tpu as pltpu
from jax.experimental.pallas import tpu_sc as plsc

EMB = 128
B_ROWS = 4096
SEQ = 50
NSPLIT = 4
B_SLICE = B_ROWS // NSPLIT
NW = 32
ROWS_W = B_SLICE // NW
NBUF = 8
DEPTH = 6
NGRP = ROWS_W // NBUF

_mesh = plsc.VectorSubcoreMesh(core_axis_name="c", subcore_axis_name="s")


@functools.partial(
    pl.kernel,
    mesh=_mesh,
    compiler_params=pltpu.CompilerParams(use_tc_tiling_on_sc=True),
    out_type=jax.ShapeDtypeStruct((B_SLICE, SEQ, EMB), jnp.float32),
    scratch_types=[
        pltpu.VMEM((ROWS_W, SEQ), jnp.int32),
    ] + [pltpu.VMEM((SEQ, EMB), jnp.float32)] * NBUF
      + [pltpu.SemaphoreType.DMA] * (2 * NBUF),
)
def _emb_lookup(ctx_hbm, table_hbm, out_hbm, idx_v, *bs):
    wid = lax.axis_index("s") * 2 + lax.axis_index("c")
    r0 = wid * ROWS_W

    bufs = bs[:NBUF]
    gs = bs[NBUF:2 * NBUF]
    ws = bs[2 * NBUF:]

    pltpu.sync_copy(ctx_hbm.at[pl.ds(r0, ROWS_W)], idx_v)

    def gather(j, b):
        pltpu.async_copy(table_hbm.at[idx_v.at[j]], bufs[b], gs[b])

    def gather_wait(j, b):
        pltpu.make_async_copy(
            table_hbm.at[idx_v.at[j]], bufs[b], gs[b]
        ).wait()

    def write(j, b):
        pltpu.async_copy(bufs[b], out_hbm.at[r0 + j], ws[b])

    def write_wait(j, b):
        pltpu.make_async_copy(bufs[b], out_hbm.at[r0 + j], ws[b]).wait()

    for j in range(DEPTH):
        gather(j, j)

    def group(jj, carry):
        j0 = jj * NBUF
        for b in range(NBUF):
            j = j0 + b
            br = (b + DEPTH) % NBUF
            gather_wait(j, b)
            write(j, b)

            @pl.when(j + DEPTH >= NBUF)
            def _():
                write_wait(j + DEPTH - NBUF, br)

            @pl.when(j + DEPTH < ROWS_W)
            def _():
                gather(j + DEPTH, br)

        return carry

    lax.fori_loop(0, NGRP, group, 0)

    for j in range(ROWS_W - (NBUF - DEPTH), ROWS_W):
        write_wait(j, j % NBUF)


def _mask_body(ctx_ref, out_ref):
    out_ref[...] = jnp.sign(ctx_ref[...])


_mask = pl.pallas_call(
    _mask_body,
    out_shape=jax.ShapeDtypeStruct((B_ROWS, SEQ), jnp.int32),
)


def kernel(context, table):
    parts = [
        _emb_lookup(context[k * B_SLICE:(k + 1) * B_SLICE], table)
        for k in range(NSPLIT)
    ]
    emb = jnp.concatenate(parts, axis=0)
    return emb, _mask(context)


# speedup vs baseline: 1.7829x; 1.7829x over previous
import functools

import jax
import jax.numpy as jnp
from jax import lax
from jax.experimental import pallas as pl
from jax.experimental.pallas import tpu as pltpu
from jax.experimental.pallas import tpu_sc as plsc

EMB = 128
B_ROWS = 4096
SEQ = 50
NW = 32
ROWS_W = B_ROWS // NW
NBUF = 8
DEPTH = 6
NGRP = ROWS_W // NBUF

_mesh = plsc.VectorSubcoreMesh(core_axis_name="c", subcore_axis_name="s")


@functools.partial(
    pl.kernel,
    mesh=_mesh,
    compiler_params=pltpu.CompilerParams(use_tc_tiling_on_sc=True, needs_layout_passes=True),
    out_type=jax.ShapeDtypeStruct((B_ROWS, SEQ, EMB), jnp.float32),
    scratch_types=[
        pltpu.VMEM((ROWS_W, SEQ), jnp.int32),
    ] + [pltpu.VMEM((SEQ, EMB), jnp.float32)] * NBUF
      + [pltpu.SemaphoreType.DMA] * (2 * NBUF),
)
def _emb_lookup(ctx_hbm, table_hbm, out_hbm, idx_v, *bs):
    wid = lax.axis_index("s") * 2 + lax.axis_index("c")
    r0 = wid * ROWS_W

    bufs = bs[:NBUF]
    gs = bs[NBUF:2 * NBUF]
    ws = bs[2 * NBUF:]

    pltpu.sync_copy(ctx_hbm.at[pl.ds(r0, ROWS_W)], idx_v)

    def gather(j, b):
        pltpu.async_copy(table_hbm.at[idx_v.at[j]], bufs[b], gs[b])

    def gather_wait(j, b):
        pltpu.make_async_copy(
            table_hbm.at[idx_v.at[j]], bufs[b], gs[b]
        ).wait()

    def write(j, b):
        pltpu.async_copy(bufs[b], out_hbm.at[r0 + j], ws[b])

    def write_wait(j, b):
        pltpu.make_async_copy(bufs[b], out_hbm.at[r0 + j], ws[b]).wait()

    for j in range(DEPTH):
        gather(j, j)

    def group(jj, carry):
        j0 = jj * NBUF
        for b in range(NBUF):
            j = j0 + b
            br = (b + DEPTH) % NBUF
            gather_wait(j, b)
            write(j, b)

            @pl.when(j + DEPTH >= NBUF)
            def _():
                write_wait(j + DEPTH - NBUF, br)

            @pl.when(j + DEPTH < ROWS_W)
            def _():
                gather(j + DEPTH, br)

        return carry

    lax.fori_loop(0, NGRP, group, 0)

    for j in range(ROWS_W - (NBUF - DEPTH), ROWS_W):
        write_wait(j, j % NBUF)


def _mask_body(ctx_ref, out_ref):
    out_ref[...] = jnp.sign(ctx_ref[...])


_mask = pl.pallas_call(
    _mask_body,
    out_shape=jax.ShapeDtypeStruct((B_ROWS, SEQ), jnp.int32),
)


def kernel(context, table):
    emb = _emb_lookup(context, table)
    return emb, _mask(context)
